# baseline (device time: 30409 ns/iter reference)
import jax
import jax.numpy as jnp
from jax import lax
from jax.experimental import pallas as pl
from jax.experimental.pallas import tpu as pltpu

N = 16
T = 512
TP = T // N
D = 512
F = 1024
E = 32
EP = E // N
PAD = 8
C = 16
MESH = pl.DeviceIdType.MESH

ORD = [0, 1, 15, 2, 14, 3, 13, 4, 12, 5, 11, 6, 10, 7, 9, 8]
POS = {off: j for j, off in enumerate(ORD)}
HALF = 8


def kernel(x, router, W1, W2):
    rpay = router.T[None]

    def body(x_ref, rpay_ref, w1_ref, w2_ref, out_ref,
             rbuf, sxbuf, rxbuf, swbuf, rwbuf, ybuf, rybuf,
             s0, r0, sx, rx, sw, rw, sy, ry):
        me = lax.axis_index("i")

        barrier = pltpu.get_barrier_semaphore()
        for off in range(1, N):
            pl.semaphore_signal(barrier, inc=1,
                                device_id=(lax.rem(me + off, N),),
                                device_id_type=MESH)
        pl.semaphore_wait(barrier, N - 1)

        rbuf[pl.ds(me, 1), pl.ds(0, EP)] = rpay_ref[...]
        sends = []
        for off in range(1, N):
            dst = lax.rem(me + off, N)
            rdma = pltpu.make_async_remote_copy(
                src_ref=rpay_ref,
                dst_ref=rbuf.at[pl.ds(me, 1), pl.ds(0, EP)],
                send_sem=s0.at[off], recv_sem=r0.at[off],
                device_id=(dst,), device_id_type=MESH)
            rdma.start()
            sends.append(rdma)
        for off in range(1, N):
            src = lax.rem(me + (N - off), N)
            pltpu.make_async_remote_copy(
                src_ref=rpay_ref,
                dst_ref=rbuf.at[pl.ds(src, 1), pl.ds(0, EP)],
                send_sem=s0.at[off], recv_sem=r0.at[off],
                device_id=(src,), device_id_type=MESH).wait_recv()

        rt = rbuf[...].reshape(N * PAD, D)
        xl = x_ref[...]
        gates = lax.dot_general(
            xl, rt, (((1,), (1,)), ((), ())),
            precision=lax.Precision.HIGHEST,
            preferred_element_type=jnp.float32)
        col = lax.broadcasted_iota(jnp.int32, (TP, N * PAD), 1)
        gates = jnp.where(col % PAD < EP, gates, -1e30)
        max1 = jnp.max(gates, axis=1, keepdims=True)
        masked = jnp.where(gates >= max1, -1e30, gates)
        max2 = jnp.max(masked, axis=1, keepdims=True)
        denom = 1.0 + jnp.exp(max2 - max1)
        w1v = 1.0 / denom
        w2v = jnp.exp(max2 - max1) / denom
        Wm = (jnp.where(gates == max1, w1v, 0.0)
              + jnp.where(gates == max2, w2v, 0.0))
        WmT = Wm.T

        ri = lax.broadcasted_iota(jnp.int32, (TP, TP), 0)
        ci = lax.broadcasted_iota(jnp.int32, (TP, TP), 1)
        U = (ri < ci).astype(jnp.float32)
        slot_io3 = lax.broadcasted_iota(jnp.int32, (N, C, TP), 1)
        lane_io3 = lax.broadcasted_iota(jnp.int32, (N, C, 128), 2)

        j_io = lax.broadcasted_iota(jnp.int32, (N, 1), 0)
        ord_col = jnp.where(j_io % 2 == 1, (j_io + 1) // 2,
                            jnp.where(j_io == 0, 0, N - j_io // 2))
        q_io = lax.broadcasted_iota(jnp.int32, (N, N), 1)
        PM = (q_io == (me + ord_col) % N).astype(jnp.float32)

        Wr = WmT.reshape(N, PAD, TP)
        W0 = jnp.dot(PM, Wr[:, 0, :],
                     preferred_element_type=jnp.float32)
        W1g = jnp.dot(PM, Wr[:, 1, :],
                      preferred_element_type=jnp.float32)
        Sel = ((W0 + W1g) > 0.0).astype(jnp.float32)
        Pos = jnp.dot(Sel, U, preferred_element_type=jnp.float32)
        P_all = ((slot_io3 == Pos[:, None, :].astype(jnp.int32))
                 .astype(jnp.float32) * Sel[:, None, :])
        xs_all = lax.dot_general(
            P_all, xl, (((2,), (0,)), ((), ())),
            preferred_element_type=jnp.float32)
        w0c = lax.dot_general(
            P_all, W0, (((2,), (1,)), ((0,), (0,))),
            preferred_element_type=jnp.float32)
        w1c = lax.dot_general(
            P_all, W1g, (((2,), (1,)), ((0,), (0,))),
            preferred_element_type=jnp.float32)
        wm_all = jnp.where(lane_io3 == 0, w0c[:, :, None],
                           jnp.where(lane_io3 == 1, w1c[:, :, None], 0.0))
        sxbuf[...] = xs_all.astype(jnp.bfloat16)
        swbuf[...] = wm_all.astype(jnp.bfloat16)

        rxbuf[pl.ds(0, 1)] = sxbuf[pl.ds(0, 1)]
        rwbuf[pl.ds(0, 1)] = swbuf[pl.ds(0, 1)]
        for j in range(N - 1, 0, -1):
            off = ORD[j]
            q = lax.rem(me + off, N)
            jr = POS[N - off]
            for src_buf, dst_buf, ssem, rsem in (
                    (sxbuf, rxbuf, sx, rx), (swbuf, rwbuf, sw, rw)):
                rdma = pltpu.make_async_remote_copy(
                    src_ref=src_buf.at[pl.ds(j, 1)],
                    dst_ref=dst_buf.at[pl.ds(jr, 1)],
                    send_sem=ssem.at[j], recv_sem=rsem.at[jr],
                    device_id=(q,), device_id_type=MESH)
                rdma.start()
                sends.append(rdma)

        w1b = [w1_ref[j].astype(jnp.bfloat16) for j in range(EP)]
        w2b = [w2_ref[j].astype(jnp.bfloat16) for j in range(EP)]
        for half in range(2):
            lo = half * HALF
            for j in range(max(lo, 1), lo + HALF):
                src = lax.rem(me + (N - ORD[j]), N)
                for buf, rsem in ((rxbuf, rx), (rwbuf, rw)):
                    pltpu.make_async_remote_copy(
                        src_ref=buf.at[pl.ds(j, 1)],
                        dst_ref=buf.at[pl.ds(j, 1)],
                        send_sem=sx.at[j], recv_sem=rsem.at[j],
                        device_id=(src,), device_id_type=MESH).wait_recv()
            Xh = rxbuf[lo:lo + HALF].reshape(HALF * C, D)
            wmh = rwbuf[lo:lo + HALF].reshape(HALF * C, 128)
            yh = jnp.zeros((HALF * C, D), jnp.float32)
            for j in range(EP):
                wj = wmh[:, j:j + 1].astype(jnp.float32)
                h = jnp.maximum(
                    jnp.dot(Xh, w1b[j],
                            preferred_element_type=jnp.float32), 0.0)
                hw = (h * wj).astype(jnp.bfloat16)
                yh = yh + jnp.dot(hw, w2b[j],
                                  preferred_element_type=jnp.float32)
            ybuf[lo:lo + HALF] = yh.reshape(HALF, C, D).astype(jnp.bfloat16)

            for j in range(max(lo, 1), lo + HALF):
                dst = lax.rem(me + ORD[j], N)
                jr = POS[N - ORD[j]]
                rdma = pltpu.make_async_remote_copy(
                    src_ref=ybuf.at[pl.ds(j, 1)],
                    dst_ref=rybuf.at[pl.ds(jr, 1)],
                    send_sem=sy.at[j], recv_sem=ry.at[jr],
                    device_id=(dst,), device_id_type=MESH)
                rdma.start()
                sends.append(rdma)
            if half == 0:
                rybuf[pl.ds(0, 1)] = ybuf[pl.ds(0, 1)]

        for j in range(1, N):
            src = lax.rem(me + ORD[j], N)
            pltpu.make_async_remote_copy(
                src_ref=ybuf.at[pl.ds(j, 1)],
                dst_ref=rybuf.at[pl.ds(j, 1)],
                send_sem=sy.at[j], recv_sem=ry.at[j],
                device_id=(src,), device_id_type=MESH).wait_recv()
        P_flat = P_all.reshape(N * C, TP).astype(jnp.bfloat16)
        Y_flat = rybuf[...].reshape(N * C, D)
        out_ref[...] = lax.dot_general(
            P_flat, Y_flat, (((0,), (0,)), ((), ())),
            preferred_element_type=jnp.float32)

        for rdma in sends:
            rdma.wait_send()

    return pl.pallas_call(
        body,
        out_shape=jax.ShapeDtypeStruct((TP, D), jnp.float32),
        in_specs=[pl.BlockSpec(memory_space=pltpu.VMEM)] * 4,
        out_specs=pl.BlockSpec(memory_space=pltpu.VMEM),
        scratch_shapes=[
            pltpu.VMEM((N, PAD, D), jnp.float32),
            pltpu.VMEM((N, C, D), jnp.bfloat16),
            pltpu.VMEM((N, C, D), jnp.bfloat16),
            pltpu.VMEM((N, C, 128), jnp.bfloat16),
            pltpu.VMEM((N, C, 128), jnp.bfloat16),
            pltpu.VMEM((N, C, D), jnp.bfloat16),
            pltpu.VMEM((N, C, D), jnp.bfloat16),
            pltpu.SemaphoreType.DMA((N,)),
            pltpu.SemaphoreType.DMA((N,)),
            pltpu.SemaphoreType.DMA((N,)),
            pltpu.SemaphoreType.DMA((N,)),
            pltpu.SemaphoreType.DMA((N,)),
            pltpu.SemaphoreType.DMA((N,)),
            pltpu.SemaphoreType.DMA((N,)),
            pltpu.SemaphoreType.DMA((N,)),
        ],
        compiler_params=pltpu.CompilerParams(collective_id=0),
    )(x, rpay, W1, W2)


# device time: 29566 ns/iter; 1.0285x vs baseline; 1.0285x over previous
import jax
import jax.numpy as jnp
from jax import lax
from jax.experimental import pallas as pl
from jax.experimental.pallas import tpu as pltpu

N = 16
T = 512
TP = T // N
D = 512
F = 1024
E = 32
EP = E // N
PAD = 8
C = 16
MESH = pl.DeviceIdType.MESH


def kernel(x, router, W1, W2):
    rpay = router.T[None]

    def body(x_ref, rpay_ref, w1_ref, w2_ref, out_ref,
             rbuf, sxbuf, rxbuf, swbuf, rwbuf, ybuf, rybuf,
             s0, r0, sx, rx, sw, rw, sy, ry):
        me = lax.axis_index("i")

        barrier = pltpu.get_barrier_semaphore()
        for off in range(1, N):
            pl.semaphore_signal(barrier, inc=1,
                                device_id=(lax.rem(me + off, N),),
                                device_id_type=MESH)
        pl.semaphore_wait(barrier, N - 1)

        rbuf[pl.ds(me, 1), pl.ds(0, EP)] = rpay_ref[...]
        sends = []
        for off in range(1, N):
            dst = lax.rem(me + off, N)
            rdma = pltpu.make_async_remote_copy(
                src_ref=rpay_ref,
                dst_ref=rbuf.at[pl.ds(me, 1), pl.ds(0, EP)],
                send_sem=s0.at[off], recv_sem=r0.at[off],
                device_id=(dst,), device_id_type=MESH)
            rdma.start()
            sends.append(rdma)
        for off in range(1, N):
            src = lax.rem(me + (N - off), N)
            pltpu.make_async_remote_copy(
                src_ref=rpay_ref,
                dst_ref=rbuf.at[pl.ds(src, 1), pl.ds(0, EP)],
                send_sem=s0.at[off], recv_sem=r0.at[off],
                device_id=(src,), device_id_type=MESH).wait_recv()

        rt = rbuf[...].reshape(N * PAD, D)
        xl = x_ref[...]
        gates = lax.dot_general(
            xl, rt, (((1,), (1,)), ((), ())),
            precision=lax.Precision.HIGHEST,
            preferred_element_type=jnp.float32)
        col = lax.broadcasted_iota(jnp.int32, (TP, N * PAD), 1)
        gates = jnp.where(col % PAD < EP, gates, -1e30)
        max1 = jnp.max(gates, axis=1, keepdims=True)
        masked = jnp.where(gates >= max1, -1e30, gates)
        max2 = jnp.max(masked, axis=1, keepdims=True)
        denom = 1.0 + jnp.exp(max2 - max1)
        w1v = 1.0 / denom
        w2v = jnp.exp(max2 - max1) / denom
        Wm = (jnp.where(gates == max1, w1v, 0.0)
              + jnp.where(gates == max2, w2v, 0.0))
        WmT = Wm.T

        ri = lax.broadcasted_iota(jnp.int32, (TP, TP), 0)
        ci = lax.broadcasted_iota(jnp.int32, (TP, TP), 1)
        U = (ri < ci).astype(jnp.float32)
        slot_io3 = lax.broadcasted_iota(jnp.int32, (N, C, TP), 1)
        lane_io3 = lax.broadcasted_iota(jnp.int32, (N, C, 128), 2)

        Wr = WmT.reshape(N, PAD, TP)
        W0 = Wr[:, 0, :]
        W1g = Wr[:, 1, :]
        Sel = ((W0 + W1g) > 0.0).astype(jnp.float32)
        Pos = jnp.dot(Sel, U, preferred_element_type=jnp.float32)
        P_all = ((slot_io3 == Pos[:, None, :].astype(jnp.int32))
                 .astype(jnp.float32) * Sel[:, None, :])
        xs_all = lax.dot_general(
            P_all, xl, (((2,), (0,)), ((), ())),
            preferred_element_type=jnp.float32)
        w0c = lax.dot_general(
            P_all, W0, (((2,), (1,)), ((0,), (0,))),
            preferred_element_type=jnp.float32)
        w1c = lax.dot_general(
            P_all, W1g, (((2,), (1,)), ((0,), (0,))),
            preferred_element_type=jnp.float32)
        wm_all = jnp.where(lane_io3 == 0, w0c[:, :, None],
                           jnp.where(lane_io3 == 1, w1c[:, :, None], 0.0))
        sxbuf[...] = xs_all.astype(jnp.bfloat16)
        swbuf[...] = wm_all.astype(jnp.bfloat16)

        rxbuf[pl.ds(me, 1)] = sxbuf[pl.ds(me, 1)]
        rwbuf[pl.ds(me, 1)] = swbuf[pl.ds(me, 1)]
        for off in range(1, N):
            q = lax.rem(me + off, N)
            for src_buf, dst_buf, ssem, rsem in (
                    (sxbuf, rxbuf, sx, rx), (swbuf, rwbuf, sw, rw)):
                rdma = pltpu.make_async_remote_copy(
                    src_ref=src_buf.at[pl.ds(q, 1)],
                    dst_ref=dst_buf.at[pl.ds(me, 1)],
                    send_sem=ssem.at[off], recv_sem=rsem.at[off],
                    device_id=(q,), device_id_type=MESH)
                rdma.start()
                sends.append(rdma)
        w1b = [w1_ref[j].astype(jnp.bfloat16) for j in range(EP)]
        w2b = [w2_ref[j].astype(jnp.bfloat16) for j in range(EP)]
        for off in range(1, N):
            src = lax.rem(me + (N - off), N)
            for buf, rsem in ((rxbuf, rx), (rwbuf, rw)):
                pltpu.make_async_remote_copy(
                    src_ref=buf.at[pl.ds(src, 1)],
                    dst_ref=buf.at[pl.ds(src, 1)],
                    send_sem=sx.at[off], recv_sem=rsem.at[off],
                    device_id=(src,), device_id_type=MESH).wait_recv()

        X = rxbuf[...].reshape(N * C, D)
        wmr = rwbuf[...].reshape(N * C, 128)
        y = jnp.zeros((N * C, D), jnp.float32)
        for j in range(EP):
            wj = wmr[:, j:j + 1].astype(jnp.float32)
            h = jnp.maximum(
                jnp.dot(X, w1b[j],
                        preferred_element_type=jnp.float32), 0.0)
            hw = (h * wj).astype(jnp.bfloat16)
            y = y + jnp.dot(hw, w2b[j],
                            preferred_element_type=jnp.float32)
        ybuf[...] = y.reshape(N, C, D).astype(jnp.bfloat16)

        rybuf[pl.ds(me, 1)] = ybuf[pl.ds(me, 1)]
        for off in range(1, N):
            dst = lax.rem(me + off, N)
            rdma = pltpu.make_async_remote_copy(
                src_ref=ybuf.at[pl.ds(dst, 1)],
                dst_ref=rybuf.at[pl.ds(me, 1)],
                send_sem=sy.at[off], recv_sem=ry.at[off],
                device_id=(dst,), device_id_type=MESH)
            rdma.start()
            sends.append(rdma)
        for off in range(1, N):
            src = lax.rem(me + (N - off), N)
            pltpu.make_async_remote_copy(
                src_ref=ybuf.at[pl.ds(src, 1)],
                dst_ref=rybuf.at[pl.ds(src, 1)],
                send_sem=sy.at[off], recv_sem=ry.at[off],
                device_id=(src,), device_id_type=MESH).wait_recv()

        P_flat = P_all.reshape(N * C, TP).astype(jnp.bfloat16)
        Y_flat = rybuf[...].reshape(N * C, D)
        out_ref[...] = lax.dot_general(
            P_flat, Y_flat, (((0,), (0,)), ((), ())),
            preferred_element_type=jnp.float32)

        for rdma in sends:
            rdma.wait_send()

    return pl.pallas_call(
        body,
        out_shape=jax.ShapeDtypeStruct((TP, D), jnp.float32),
        in_specs=[pl.BlockSpec(memory_space=pltpu.VMEM)] * 4,
        out_specs=pl.BlockSpec(memory_space=pltpu.VMEM),
        scratch_shapes=[
            pltpu.VMEM((N, PAD, D), jnp.float32),
            pltpu.VMEM((N, C, D), jnp.bfloat16),
            pltpu.VMEM((N, C, D), jnp.bfloat16),
            pltpu.VMEM((N, C, 128), jnp.bfloat16),
            pltpu.VMEM((N, C, 128), jnp.bfloat16),
            pltpu.VMEM((N, C, D), jnp.bfloat16),
            pltpu.VMEM((N, C, D), jnp.bfloat16),
            pltpu.SemaphoreType.DMA((N,)),
            pltpu.SemaphoreType.DMA((N,)),
            pltpu.SemaphoreType.DMA((N,)),
            pltpu.SemaphoreType.DMA((N,)),
            pltpu.SemaphoreType.DMA((N,)),
            pltpu.SemaphoreType.DMA((N,)),
            pltpu.SemaphoreType.DMA((N,)),
            pltpu.SemaphoreType.DMA((N,)),
        ],
        compiler_params=pltpu.CompilerParams(collective_id=0),
    )(x, rpay, W1, W2)
